# baseline (speedup 1.0000x reference)
import functools
import math

import jax
import jax.numpy as jnp
from jax import lax
from jax.experimental import pallas as pl
from jax.experimental.pallas import tpu as pltpu
from jax.experimental.pallas import tpu_sc as plsc

D_MODEL = 64
SCALE = math.sqrt(D_MODEL)

NUM_CORES = 2
NUM_SUBCORES = 16
NUM_WORKERS = NUM_CORES * NUM_SUBCORES
LANES = 128
NBUF = 2


@functools.partial(jax.jit, static_argnums=(2, 3))
def _emb_lookup(xt, tbl, seq, batch):
  n_btile = batch // LANES
  mesh = plsc.VectorSubcoreMesh(core_axis_name="c", subcore_axis_name="s")

  scratch = [
      pltpu.VMEM((seq, LANES), jnp.int32),
      pltpu.VMEM((seq, LANES), jnp.int32),
      pltpu.VMEM((seq, LANES), jnp.int32),
  ]
  scratch += [pltpu.VMEM((LANES, 128), jnp.float32) for _ in range(NBUF)]
  scratch += [
      pltpu.VMEM((D_MODEL // 8, 8, LANES), jnp.float32) for _ in range(NBUF)
  ]
  scratch += [pltpu.SemaphoreType.DMA for _ in range(2 * NBUF)]

  @functools.partial(
      pl.kernel,
      mesh=mesh,
      out_type=jax.ShapeDtypeStruct(
          (seq, D_MODEL // 8, n_btile, 8, LANES), jnp.float32),
      scratch_types=scratch,
      compiler_params=pltpu.CompilerParams(needs_layout_passes=False),
  )
  def k(xt_hbm, tbl_hbm, out_hbm, idx_v, idxh_v, p64_v, *bufs_and_sems):
    in_bufs = bufs_and_sems[:NBUF]
    out_bufs = bufs_and_sems[NBUF:2 * NBUF]
    g_sems = bufs_and_sems[2 * NBUF:3 * NBUF]
    s_sems = bufs_and_sems[3 * NBUF:4 * NBUF]
    wid = lax.axis_index("s") * NUM_CORES + lax.axis_index("c")

    pltpu.sync_copy(xt_hbm.at[:, pl.ds(wid * LANES, LANES)], idx_v)

    def idx_prep(s, carry):
      for kk in range(LANES // 16):
        sl = (s, pl.ds(kk * 16, 16))
        v = idx_v[sl]
        idxh_v[sl] = v >> 1
        p64_v[sl] = (v & 1) << 6
      return carry

    lax.fori_loop(0, seq, idx_prep, 0, unroll=False)

    def fire_gather(s, b):
      pltpu.async_copy(tbl_hbm.at[idxh_v.at[s]], in_bufs[b], g_sems[b])

    def wait_gather(s, b):
      pltpu.make_async_copy(
          tbl_hbm.at[idxh_v.at[s]], in_bufs[b], g_sems[b]).wait()

    def fire_scatter(s, b):
      pltpu.async_copy(out_bufs[b], out_hbm.at[s, :, wid], s_sems[b])

    def wait_scatter(s, b):
      pltpu.make_async_copy(
          out_bufs[b], out_hbm.at[s, :, wid], s_sems[b]).wait()

    def transpose_scale(s, b):
      src, dst = in_bufs[b], out_bufs[b]
      for l0 in range(0, LANES, 16):
        rows = lax.iota(jnp.int32, 16) + l0
        p64 = p64_v[s, pl.ds(l0, 16)]

        def body(r8, col):
          for j in range(8):
            val = plsc.load_gather(src, [rows, col])
            dst[r8, j, pl.ds(l0, 16)] = val * SCALE
            col = col + 1
          return col

        lax.fori_loop(0, D_MODEL // 8, body, p64, unroll=False)

    for b in range(NBUF):
      fire_gather(b, b)

    for b in range(NBUF):
      wait_gather(b, b)
      transpose_scale(b, b)
      fire_gather(NBUF + b, b)
      fire_scatter(b, b)

    def outer(i, carry):
      s0 = i * NBUF
      for b in range(NBUF):
        wait_gather(s0 + b, b)
        wait_scatter(s0 - NBUF + b, b)
        transpose_scale(s0 + b, b)
        fire_gather(s0 + NBUF + b, b)
        fire_scatter(s0 + b, b)
      return carry

    lax.fori_loop(1, seq // NBUF - 1, outer, 0, unroll=False)

    s0 = seq - NBUF
    for b in range(NBUF):
      wait_gather(s0 + b, b)
      wait_scatter(s0 - NBUF + b, b)
      transpose_scale(s0 + b, b)
      fire_scatter(s0 + b, b)
    for b in range(NBUF):
      wait_scatter(s0 + b, b)

  return k(xt, tbl)


def kernel(x, emb_table):
  batch, seq = x.shape
  assert batch % (NUM_WORKERS * 128) == 0 and seq % NBUF == 0
  xt = x.astype(jnp.int32).T
  tbl = emb_table.reshape(emb_table.shape[0] // 2, 128)
  out5 = _emb_lookup(xt, tbl, seq, batch)
  return out5.transpose(2, 4, 0, 1, 3).reshape(batch, seq, D_MODEL)


# speedup vs baseline: 1.5255x; 1.5255x over previous
import functools
import math

import jax
import jax.numpy as jnp
from jax import lax
from jax.experimental import pallas as pl
from jax.experimental.pallas import tpu as pltpu
from jax.experimental.pallas import tpu_sc as plsc

D_MODEL = 64
SCALE = math.sqrt(D_MODEL)

NUM_CORES = 2
NUM_SUBCORES = 16
NUM_WORKERS = NUM_CORES * NUM_SUBCORES
LANES = 128
NBUF = 2


@functools.partial(jax.jit, static_argnums=(2, 3))
def _emb_lookup(xt, tbl, seq, batch):
  n_btile = batch // LANES
  mesh = plsc.VectorSubcoreMesh(core_axis_name="c", subcore_axis_name="s")

  scratch = [
      pltpu.VMEM((seq, LANES), jnp.int32),
      pltpu.VMEM((seq, LANES), jnp.int32),
      pltpu.VMEM((seq, LANES), jnp.int32),
  ]
  scratch += [pltpu.VMEM((LANES, 128), jnp.float32) for _ in range(NBUF)]
  scratch += [
      pltpu.VMEM((D_MODEL // 8, 8, LANES), jnp.float32) for _ in range(NBUF)
  ]
  scratch += [pltpu.SemaphoreType.DMA for _ in range(2 * NBUF)]

  @functools.partial(
      pl.kernel,
      mesh=mesh,
      out_type=jax.ShapeDtypeStruct(
          (seq, D_MODEL // 8, n_btile, 8, LANES), jnp.float32),
      scratch_types=scratch,
      compiler_params=pltpu.CompilerParams(needs_layout_passes=False),
  )
  def k(xt_hbm, tbl_hbm, out_hbm, idx_v, idxh_v, p64_v, *bufs_and_sems):
    in_bufs = bufs_and_sems[:NBUF]
    out_bufs = bufs_and_sems[NBUF:2 * NBUF]
    g_sems = bufs_and_sems[2 * NBUF:3 * NBUF]
    s_sems = bufs_and_sems[3 * NBUF:4 * NBUF]
    wid = lax.axis_index("s") * NUM_CORES + lax.axis_index("c")

    pltpu.sync_copy(xt_hbm.at[:, pl.ds(wid * LANES, LANES)], idx_v)

    def idx_prep(s, carry):
      for kk in range(LANES // 16):
        sl = (s, pl.ds(kk * 16, 16))
        v = idx_v[sl]
        idxh_v[sl] = v >> 1
        p64_v[sl] = (v & 1) << 6
      return carry

    lax.fori_loop(0, seq, idx_prep, 0, unroll=False)

    def fire_gather(s, b):
      pltpu.async_copy(tbl_hbm.at[idxh_v.at[s]], in_bufs[b], g_sems[b])

    def wait_gather(s, b):
      pltpu.make_async_copy(
          tbl_hbm.at[idxh_v.at[s]], in_bufs[b], g_sems[b]).wait()

    def fire_scatter(s, b):
      pltpu.async_copy(out_bufs[b], out_hbm.at[s, :, wid], s_sems[b])

    def wait_scatter(s, b):
      pltpu.make_async_copy(
          out_bufs[b], out_hbm.at[s, :, wid], s_sems[b]).wait()

    def transpose_scale(s, b):
      src, dst = in_bufs[b], out_bufs[b]
      for l0 in range(0, LANES, 16):
        rows = lax.iota(jnp.int32, 16) + l0
        p64 = p64_v[s, pl.ds(l0, 16)]

        @plsc.parallel_loop(0, D_MODEL // 8, unroll=2, carry=p64)
        def body(r8, col):
          for j in range(8):
            val = plsc.load_gather(src, [rows, col])
            dst[r8, j, pl.ds(l0, 16)] = val * SCALE
            col = col + 1
          return col

    for b in range(NBUF):
      fire_gather(b, b)

    for b in range(NBUF):
      wait_gather(b, b)
      transpose_scale(b, b)
      fire_gather(NBUF + b, b)
      fire_scatter(b, b)

    def outer(i, carry):
      s0 = i * NBUF
      for b in range(NBUF):
        wait_gather(s0 + b, b)
        wait_scatter(s0 - NBUF + b, b)
        transpose_scale(s0 + b, b)
        fire_gather(s0 + NBUF + b, b)
        fire_scatter(s0 + b, b)
      return carry

    lax.fori_loop(1, seq // NBUF - 1, outer, 0, unroll=False)

    s0 = seq - NBUF
    for b in range(NBUF):
      wait_gather(s0 + b, b)
      wait_scatter(s0 - NBUF + b, b)
      transpose_scale(s0 + b, b)
      fire_scatter(s0 + b, b)
    for b in range(NBUF):
      wait_scatter(s0 + b, b)

  return k(xt, tbl)


def kernel(x, emb_table):
  batch, seq = x.shape
  assert batch % (NUM_WORKERS * 128) == 0 and seq % NBUF == 0
  xt = x.astype(jnp.int32).T
  tbl = emb_table.reshape(emb_table.shape[0] // 2, 128)
  out5 = _emb_lookup(xt, tbl, seq, batch)
  return out5.transpose(2, 4, 0, 1, 3).reshape(batch, seq, D_MODEL)
